# Initial kernel scaffold; baseline (speedup 1.0000x reference)
import functools

import jax
import jax.numpy as jnp
from jax import lax
from jax.experimental import pallas as pl
from jax.experimental.pallas import tpu as pltpu
from jax.experimental.pallas import tpu_sc as plsc

IN_DIM = 1024
H1 = 1024
H2 = 512
H3 = 256
LATENT = 256
D = 32
K = 8192
BETA = 0.25
B = 1024

N_CODES = B * (LATENT // D)

BT = 256
NBT = B // BT
QT = 512
NQT = N_CODES // QT
KC = 2048
NKC = K // KC

SC_NC = 2
SC_NS = 16
SC_NW = SC_NC * SC_NS
ROWS_PER_W = N_CODES // SC_NW


def _lrelu(a):
    return jnp.where(a >= 0, a, 0.01 * a)


def _encoder_body(x_ref, w1_ref, b1_ref, w2_ref, b2_ref, w3_ref, b3_ref,
                  w4_ref, b4_ref, z_ref):
    h = _lrelu(jnp.dot(x_ref[...], w1_ref[...],
                       preferred_element_type=jnp.float32) + b1_ref[...])
    h = _lrelu(jnp.dot(h, w2_ref[...],
                       preferred_element_type=jnp.float32) + b2_ref[...])
    h = _lrelu(jnp.dot(h, w3_ref[...],
                       preferred_element_type=jnp.float32) + b3_ref[...])
    z_ref[...] = jnp.dot(h, w4_ref[...],
                         preferred_element_type=jnp.float32) + b4_ref[...]


def _encoder(x, eW1, eb1, eW2, eb2, eW3, eb3, eW4, eb4):
    const = lambda shape: pl.BlockSpec(shape, lambda i: (0, 0))
    return pl.pallas_call(
        _encoder_body,
        grid=(NBT,),
        in_specs=[
            pl.BlockSpec((BT, IN_DIM), lambda i: (i, 0)),
            const((IN_DIM, H1)), const((1, H1)),
            const((H1, H2)), const((1, H2)),
            const((H2, H3)), const((1, H3)),
            const((H3, LATENT)), const((1, LATENT)),
        ],
        out_specs=pl.BlockSpec((BT, LATENT), lambda i: (i, 0)),
        out_shape=jax.ShapeDtypeStruct((B, LATENT), jnp.float32),
    )(x, eW1, eb1.reshape(1, H1), eW2, eb2.reshape(1, H2),
      eW3, eb3.reshape(1, H3), eW4, eb4.reshape(1, LATENT))


def _argmin_body(r_ref, embT_ref, idx_ref):
    r = r_ref[...]
    best_val = None
    best_idx = None
    for c in range(NKC):
        embT_c = embT_ref[:, c * KC:(c + 1) * KC]
        esq = jnp.sum(embT_c * embT_c, axis=0, keepdims=True)
        s = esq - 2.0 * jnp.dot(r, embT_c,
                                preferred_element_type=jnp.float32)
        cmin = jnp.min(s, axis=1, keepdims=True)
        iota = lax.broadcasted_iota(jnp.int32, (QT, KC), 1) + c * KC
        cidx = jnp.min(jnp.where(s <= cmin, iota, K),
                       axis=1, keepdims=True)
        if c == 0:
            best_val, best_idx = cmin, cidx
        else:
            better = cmin < best_val
            best_idx = jnp.where(better, cidx, best_idx)
            best_val = jnp.minimum(cmin, best_val)
    idx_ref[...] = best_idx


def _vq_argmin(r, embT):
    return pl.pallas_call(
        _argmin_body,
        grid=(NQT,),
        in_specs=[
            pl.BlockSpec((QT, D), lambda i: (i, 0)),
            pl.BlockSpec((D, K), lambda i: (0, 0)),
        ],
        out_specs=pl.BlockSpec((QT, 1), lambda i: (i, 0)),
        out_shape=jax.ShapeDtypeStruct((N_CODES, 1), jnp.int32),
    )(r, embT)


def _sc_gather_body(table_hbm, idx_hbm, out_hbm, idx_v, rows_v, sem):
    wid = lax.axis_index("s") * SC_NC + lax.axis_index("c")
    base = wid * ROWS_PER_W
    pltpu.sync_copy(idx_hbm.at[pl.ds(base, ROWS_PER_W)], idx_v)
    pltpu.async_copy(table_hbm.at[idx_v], rows_v, sem).wait()
    pltpu.sync_copy(rows_v, out_hbm.at[pl.ds(base, ROWS_PER_W)])


def _sc_gather(emb, inds):
    mesh = plsc.VectorSubcoreMesh(core_axis_name="c", subcore_axis_name="s")
    k = pl.kernel(
        _sc_gather_body,
        out_type=jax.ShapeDtypeStruct((N_CODES, D), jnp.float32),
        mesh=mesh,
        scratch_types=[
            pltpu.VMEM((ROWS_PER_W,), jnp.int32),
            pltpu.VMEM((ROWS_PER_W, D), jnp.float32),
            pltpu.SemaphoreType.DMA,
        ],
    )
    return k(emb, inds)


def _decoder_body(z_ref, q_ref, x_ref, w1_ref, b1_ref, w2_ref, b2_ref,
                  w3_ref, b3_ref, w4_ref, b4_ref,
                  qst_ref, recon_ref, loss_ref, rl_ref, vq_ref):
    i = pl.program_id(0)
    zt = z_ref[...]
    dq = q_ref[...] - zt
    qst = zt + dq
    qst_ref[...] = qst
    vq_part = ((1.0 + BETA) * jnp.sum(dq * dq))[None, None]

    h = _lrelu(jnp.dot(qst, w1_ref[...],
                       preferred_element_type=jnp.float32) + b1_ref[...])
    h = _lrelu(jnp.dot(h, w2_ref[...],
                       preferred_element_type=jnp.float32) + b2_ref[...])
    h = _lrelu(jnp.dot(h, w3_ref[...],
                       preferred_element_type=jnp.float32) + b3_ref[...])
    logits = jnp.dot(h, w4_ref[...],
                     preferred_element_type=jnp.float32) + b4_ref[...]
    recon = 1.0 / (1.0 + jnp.exp(-logits))
    recon_ref[...] = recon

    rd = recon - x_ref[...]
    rl_part = jnp.sum(rd * rd)[None, None]

    @pl.when(i == 0)
    def _init():
        rl_ref[...] = rl_part
        vq_ref[...] = vq_part

    @pl.when(i > 0)
    def _acc():
        rl_ref[...] = rl_ref[...] + rl_part
        vq_ref[...] = vq_ref[...] + vq_part

    @pl.when(i == NBT - 1)
    def _finalize():
        rl = rl_ref[...] * (1.0 / B)
        vq = vq_ref[...] * (1.0 / B)
        rl_ref[...] = rl
        vq_ref[...] = vq
        loss_ref[...] = rl + vq


def _decoder(z, q, x, dW1, db1, dW2, db2, dW3, db3, dW4, db4):
    const = lambda shape: pl.BlockSpec(shape, lambda i: (0, 0))
    return pl.pallas_call(
        _decoder_body,
        grid=(NBT,),
        in_specs=[
            pl.BlockSpec((BT, LATENT), lambda i: (i, 0)),
            pl.BlockSpec((BT, LATENT), lambda i: (i, 0)),
            pl.BlockSpec((BT, IN_DIM), lambda i: (i, 0)),
            const((LATENT, H3)), const((1, H3)),
            const((H3, H2)), const((1, H2)),
            const((H2, H1)), const((1, H1)),
            const((H1, IN_DIM)), const((1, IN_DIM)),
        ],
        out_specs=[
            pl.BlockSpec((BT, LATENT), lambda i: (i, 0)),
            pl.BlockSpec((BT, IN_DIM), lambda i: (i, 0)),
            const((1, 1)), const((1, 1)), const((1, 1)),
        ],
        out_shape=[
            jax.ShapeDtypeStruct((B, LATENT), jnp.float32),
            jax.ShapeDtypeStruct((B, IN_DIM), jnp.float32),
            jax.ShapeDtypeStruct((1, 1), jnp.float32),
            jax.ShapeDtypeStruct((1, 1), jnp.float32),
            jax.ShapeDtypeStruct((1, 1), jnp.float32),
        ],
    )(z, q, x, dW1, db1.reshape(1, H3), dW2, db2.reshape(1, H2),
      dW3, db3.reshape(1, H1), dW4, db4.reshape(1, IN_DIM))


def kernel(x, eW1, eb1, eW2, eb2, eW3, eb3, eW4, eb4,
           dW1, db1, dW2, db2, dW3, db3, dW4, db4, emb):
    z = _encoder(x, eW1, eb1, eW2, eb2, eW3, eb3, eW4, eb4)
    r = z.reshape(N_CODES, D)
    inds = _vq_argmin(r, emb.T)
    q = _sc_gather(emb, inds.reshape(N_CODES)).reshape(B, LATENT)
    q_st, recon, loss, recon_loss, vq_loss = _decoder(
        z, q, x, dW1, db1, dW2, db2, dW3, db3, dW4, db4)
    return (z, q_st, recon,
            loss.reshape(()), recon_loss.reshape(()), vq_loss.reshape(()))


# speedup vs baseline: 7.8295x; 7.8295x over previous
import functools

import jax
import jax.numpy as jnp
from jax import lax
from jax.experimental import pallas as pl
from jax.experimental.pallas import tpu as pltpu
from jax.experimental.pallas import tpu_sc as plsc

IN_DIM = 1024
H1 = 1024
H2 = 512
H3 = 256
LATENT = 256
D = 32
K = 8192
BETA = 0.25
B = 1024

N_CODES = B * (LATENT // D)

BT = 256
NBT = B // BT
QT = 512
NQT = N_CODES // QT
KC = 2048
NKC = K // KC

SC_NC = 2
SC_NS = 16
SC_NW = SC_NC * SC_NS
ROWS_PER_W = N_CODES // SC_NW


def _lrelu(a):
    return jnp.where(a >= 0, a, 0.01 * a)


def _encoder_body(x_ref, w1_ref, b1_ref, w2_ref, b2_ref, w3_ref, b3_ref,
                  w4_ref, b4_ref, z_ref):
    h = _lrelu(jnp.dot(x_ref[...], w1_ref[...],
                       preferred_element_type=jnp.float32) + b1_ref[...])
    h = _lrelu(jnp.dot(h, w2_ref[...],
                       preferred_element_type=jnp.float32) + b2_ref[...])
    h = _lrelu(jnp.dot(h, w3_ref[...],
                       preferred_element_type=jnp.float32) + b3_ref[...])
    z_ref[...] = jnp.dot(h, w4_ref[...],
                         preferred_element_type=jnp.float32) + b4_ref[...]


def _encoder(x, eW1, eb1, eW2, eb2, eW3, eb3, eW4, eb4):
    const = lambda shape: pl.BlockSpec(shape, lambda i: (0, 0))
    return pl.pallas_call(
        _encoder_body,
        grid=(NBT,),
        in_specs=[
            pl.BlockSpec((BT, IN_DIM), lambda i: (i, 0)),
            const((IN_DIM, H1)), const((1, H1)),
            const((H1, H2)), const((1, H2)),
            const((H2, H3)), const((1, H3)),
            const((H3, LATENT)), const((1, LATENT)),
        ],
        out_specs=pl.BlockSpec((BT, LATENT), lambda i: (i, 0)),
        out_shape=jax.ShapeDtypeStruct((B, LATENT), jnp.float32),
    )(x, eW1, eb1.reshape(1, H1), eW2, eb2.reshape(1, H2),
      eW3, eb3.reshape(1, H3), eW4, eb4.reshape(1, LATENT))


def _argmin_body(r_ref, emb_ref, rsq_ref, esq_ref, idx_ref):
    r = r_ref[...]
    rsq = rsq_ref[...]
    best_val = None
    best_idx = None
    for c in range(NKC):
        emb_c = emb_ref[c * KC:(c + 1) * KC, :]
        esq = esq_ref[:, c * KC:(c + 1) * KC]
        s = (rsq + esq) - 2.0 * lax.dot_general(
            r, emb_c, (((1,), (1,)), ((), ())),
            preferred_element_type=jnp.float32)
        cmin = jnp.min(s, axis=1, keepdims=True)
        iota = lax.broadcasted_iota(jnp.int32, (QT, KC), 1) + c * KC
        cidx = jnp.min(jnp.where(s <= cmin, iota, K),
                       axis=1, keepdims=True)
        if c == 0:
            best_val, best_idx = cmin, cidx
        else:
            better = cmin < best_val
            best_idx = jnp.where(better, cidx, best_idx)
            best_val = jnp.minimum(cmin, best_val)
    idx_ref[...] = best_idx


def _vq_argmin(r, emb, rsq, esq):
    return pl.pallas_call(
        _argmin_body,
        grid=(NQT,),
        in_specs=[
            pl.BlockSpec((QT, D), lambda i: (i, 0)),
            pl.BlockSpec((K, D), lambda i: (0, 0)),
            pl.BlockSpec((QT, 1), lambda i: (i, 0)),
            pl.BlockSpec((1, K), lambda i: (0, 0)),
        ],
        out_specs=pl.BlockSpec((QT, 1), lambda i: (i, 0)),
        out_shape=jax.ShapeDtypeStruct((N_CODES, 1), jnp.int32),
    )(r, emb, rsq, esq)


def _sc_gather_body(table_hbm, idx_hbm, out_hbm, idx_v, rows_v, sem):
    wid = lax.axis_index("s") * SC_NC + lax.axis_index("c")
    base = wid * ROWS_PER_W
    pltpu.sync_copy(idx_hbm.at[pl.ds(base, ROWS_PER_W)], idx_v)
    pltpu.async_copy(table_hbm.at[idx_v], rows_v, sem).wait()
    pltpu.sync_copy(rows_v, out_hbm.at[pl.ds(base, ROWS_PER_W)])


def _sc_gather(emb, inds):
    mesh = plsc.VectorSubcoreMesh(core_axis_name="c", subcore_axis_name="s")
    k = pl.kernel(
        _sc_gather_body,
        out_type=jax.ShapeDtypeStruct((N_CODES, D), jnp.float32),
        mesh=mesh,
        compiler_params=pltpu.CompilerParams(use_tc_tiling_on_sc=False),
        scratch_types=[
            pltpu.VMEM((ROWS_PER_W,), jnp.int32),
            pltpu.VMEM((ROWS_PER_W, D), jnp.float32),
            pltpu.SemaphoreType.DMA,
        ],
    )
    return k(emb, inds)


def _decoder_body(z_ref, q_ref, x_ref, w1_ref, b1_ref, w2_ref, b2_ref,
                  w3_ref, b3_ref, w4_ref, b4_ref,
                  qst_ref, recon_ref, loss_ref, rl_ref, vq_ref):
    i = pl.program_id(0)
    zt = z_ref[...]
    dq = q_ref[...] - zt
    qst = zt + dq
    qst_ref[...] = qst
    vq_part = ((1.0 + BETA) * jnp.sum(dq * dq))[None, None]

    h = _lrelu(jnp.dot(qst, w1_ref[...],
                       preferred_element_type=jnp.float32) + b1_ref[...])
    h = _lrelu(jnp.dot(h, w2_ref[...],
                       preferred_element_type=jnp.float32) + b2_ref[...])
    h = _lrelu(jnp.dot(h, w3_ref[...],
                       preferred_element_type=jnp.float32) + b3_ref[...])
    logits = jnp.dot(h, w4_ref[...],
                     preferred_element_type=jnp.float32) + b4_ref[...]
    recon = 1.0 / (1.0 + jnp.exp(-logits))
    recon_ref[...] = recon

    rd = recon - x_ref[...]
    rl_part = jnp.sum(rd * rd)[None, None]

    @pl.when(i == 0)
    def _init():
        rl_ref[...] = rl_part
        vq_ref[...] = vq_part

    @pl.when(i > 0)
    def _acc():
        rl_ref[...] = rl_ref[...] + rl_part
        vq_ref[...] = vq_ref[...] + vq_part

    @pl.when(i == NBT - 1)
    def _finalize():
        rl = rl_ref[...] * (1.0 / B)
        vq = vq_ref[...] * (1.0 / B)
        rl_ref[...] = rl
        vq_ref[...] = vq
        loss_ref[...] = rl + vq


def _decoder(z, q, x, dW1, db1, dW2, db2, dW3, db3, dW4, db4):
    const = lambda shape: pl.BlockSpec(shape, lambda i: (0, 0))
    return pl.pallas_call(
        _decoder_body,
        grid=(NBT,),
        in_specs=[
            pl.BlockSpec((BT, LATENT), lambda i: (i, 0)),
            pl.BlockSpec((BT, LATENT), lambda i: (i, 0)),
            pl.BlockSpec((BT, IN_DIM), lambda i: (i, 0)),
            const((LATENT, H3)), const((1, H3)),
            const((H3, H2)), const((1, H2)),
            const((H2, H1)), const((1, H1)),
            const((H1, IN_DIM)), const((1, IN_DIM)),
        ],
        out_specs=[
            pl.BlockSpec((BT, LATENT), lambda i: (i, 0)),
            pl.BlockSpec((BT, IN_DIM), lambda i: (i, 0)),
            const((1, 1)), const((1, 1)), const((1, 1)),
        ],
        out_shape=[
            jax.ShapeDtypeStruct((B, LATENT), jnp.float32),
            jax.ShapeDtypeStruct((B, IN_DIM), jnp.float32),
            jax.ShapeDtypeStruct((1, 1), jnp.float32),
            jax.ShapeDtypeStruct((1, 1), jnp.float32),
            jax.ShapeDtypeStruct((1, 1), jnp.float32),
        ],
    )(z, q, x, dW1, db1.reshape(1, H3), dW2, db2.reshape(1, H2),
      dW3, db3.reshape(1, H1), dW4, db4.reshape(1, IN_DIM))


def kernel(x, eW1, eb1, eW2, eb2, eW3, eb3, eW4, eb4,
           dW1, db1, dW2, db2, dW3, db3, dW4, db4, emb):
    z = _encoder(x, eW1, eb1, eW2, eb2, eW3, eb3, eW4, eb4)
    r = z.reshape(N_CODES, D)
    rsq = jnp.sum(r ** 2, axis=1, keepdims=True)
    esq = jnp.sum(emb ** 2, axis=1).reshape(1, K)
    esq_t = jnp.sum(emb ** 2, axis=1)
    d_def = (rsq + esq_t) - 2.0 * (r @ emb.T)
    inds = jnp.argmin(d_def, axis=1).astype(jnp.int32).reshape(N_CODES, 1)
    q = _sc_gather(emb, inds.reshape(N_CODES)).reshape(B, LATENT)
    q_st, recon, loss, recon_loss, vq_loss = _decoder(
        z, q, x, dW1, db1, dW2, db2, dW3, db3, dW4, db4)
    return (z, q_st, recon,
            loss.reshape(()), recon_loss.reshape(()), vq_loss.reshape(()))
